# Initial kernel scaffold; baseline (speedup 1.0000x reference)
import jax
import jax.numpy as jnp
from jax.experimental import pallas as pl


def kernel(q_points, s_points, s_feats, neighbor_indices, W1, b1, g1, be1, W2, b2, g2, be2, W3, b3, g3, be3):
    raise NotImplementedError("write your pallas kernel here")


# speedup vs baseline: 479.1820x; 479.1820x over previous
import functools

import jax
import jax.numpy as jnp
from jax import lax
from jax.experimental import pallas as pl
from jax.experimental.pallas import tpu as pltpu
from jax.experimental.pallas import tpu_sc as plsc

EPS = 1e-5
NW = 32
GCH = 1024
GSUB = 128


def _sc_gather(table, idx):
    R, C = table.shape
    S = idx.shape[0]
    per_w = S // NW
    ngroups = per_w // GCH
    nsub = GCH // GSUB

    mesh = plsc.VectorSubcoreMesh(core_axis_name="c", subcore_axis_name="s")

    @functools.partial(
        pl.kernel,
        out_type=jax.ShapeDtypeStruct((S, C), jnp.float32),
        mesh=mesh,
        compiler_params=pltpu.CompilerParams(use_tc_tiling_on_sc=False),
        scratch_types=[
            pltpu.VMEM((per_w,), jnp.int32),
            pltpu.VMEM((GCH, C), jnp.float32),
            pltpu.SemaphoreType.DMA,
        ],
    )
    def gk(table_hbm, idx_hbm, out_hbm, idx_v, rows_v, sem):
        wid = lax.axis_index("s") * 2 + lax.axis_index("c")
        base = wid * per_w
        pltpu.sync_copy(idx_hbm.at[pl.ds(base, per_w)], idx_v)

        def body(c, carry):
            cb = c * GCH
            handles = []
            for j in range(nsub):
                handles.append(pltpu.async_copy(
                    table_hbm.at[idx_v.at[pl.ds(cb + j * GSUB, GSUB)]],
                    rows_v.at[pl.ds(j * GSUB, GSUB)],
                    sem))
            for h in handles:
                h.wait()
            pltpu.sync_copy(rows_v, out_hbm.at[pl.ds(base + cb, GCH)])
            return carry

        lax.fori_loop(0, ngroups, body, 0)

    return gk(table, idx)


def _stats_outputs(Cout):
    return (
        pl.BlockSpec((2, Cout), lambda i: (0, 0)),
        jax.ShapeDtypeStruct((2, Cout), jnp.float32),
    )


def _accum_stats(st_ref, y):
    @pl.when(pl.program_id(0) == 0)
    def _():
        st_ref[...] = jnp.zeros_like(st_ref)

    st_ref[...] += jnp.concatenate(
        [jnp.sum(y, axis=0, keepdims=True),
         jnp.sum(y * y, axis=0, keepdims=True)], axis=0)


def _stage1_pass(x, qpad, w1t, b1, blk):
    S, Cin = x.shape
    Cout = w1t.shape[1]
    grid = (S // blk,)
    nq = qpad.shape[0] // blk

    def body(x_ref, p_ref, w_ref, b_ref, y_ref, st_ref):
        h = x_ref[...] - p_ref[...]
        y = jnp.dot(h, w_ref[...], preferred_element_type=jnp.float32) + b_ref[...]
        y_ref[...] = y
        _accum_stats(st_ref, y)

    st_spec, st_shape = _stats_outputs(Cout)
    return pl.pallas_call(
        body,
        grid=grid,
        in_specs=[
            pl.BlockSpec((blk, Cin), lambda i: (i, 0)),
            pl.BlockSpec((blk, Cin), lambda i: (i % nq, 0)),
            pl.BlockSpec((Cin, Cout), lambda i: (0, 0)),
            pl.BlockSpec((1, Cout), lambda i: (0, 0)),
        ],
        out_specs=[pl.BlockSpec((blk, Cout), lambda i: (i, 0)), st_spec],
        out_shape=[jax.ShapeDtypeStruct((S, Cout), jnp.float32), st_shape],
    )(x, qpad, w1t, b1)


def _stage2_pass(y1, a1, c1, w2t, b2, blk):
    S, Cin = y1.shape
    Cout = w2t.shape[1]
    grid = (S // blk,)

    def body(x_ref, a_ref, c_ref, w_ref, b_ref, y_ref, st_ref):
        h = jnp.maximum(x_ref[...] * a_ref[...] + c_ref[...], 0.0)
        y = jnp.dot(h, w_ref[...], preferred_element_type=jnp.float32) + b_ref[...]
        y_ref[...] = y
        _accum_stats(st_ref, y)

    st_spec, st_shape = _stats_outputs(Cout)
    return pl.pallas_call(
        body,
        grid=grid,
        in_specs=[
            pl.BlockSpec((blk, Cin), lambda i: (i, 0)),
            pl.BlockSpec((1, Cin), lambda i: (0, 0)),
            pl.BlockSpec((1, Cin), lambda i: (0, 0)),
            pl.BlockSpec((Cin, Cout), lambda i: (0, 0)),
            pl.BlockSpec((1, Cout), lambda i: (0, 0)),
        ],
        out_specs=[pl.BlockSpec((blk, Cout), lambda i: (i, 0)), st_spec],
        out_shape=[jax.ShapeDtypeStruct((S, Cout), jnp.float32), st_shape],
    )(y1, a1, c1, w2t, b2)


def _final_pass(y2_3d, a2, c2, w3t, b3, blkj):
    K, J, Cin = y2_3d.shape
    Cout = w3t.shape[1]
    grid = (J // blkj,)

    def body(y_ref, a_ref, c_ref, w_ref, b_ref, z_ref, st_ref):
        h = jnp.maximum(y_ref[...] * a_ref[...] + c_ref[...], 0.0)
        y3 = jnp.dot(h.reshape(K * blkj, Cin), w_ref[...],
                     preferred_element_type=jnp.float32) + b_ref[...]
        z_ref[...] = jnp.max(y3.reshape(K, blkj, Cout), axis=0)
        _accum_stats(st_ref, y3)

    return pl.pallas_call(
        body,
        grid=grid,
        in_specs=[
            pl.BlockSpec((K, blkj, Cin), lambda i: (0, i, 0)),
            pl.BlockSpec((1, Cin), lambda i: (0, 0)),
            pl.BlockSpec((1, Cin), lambda i: (0, 0)),
            pl.BlockSpec((Cin, Cout), lambda i: (0, 0)),
            pl.BlockSpec((1, Cout), lambda i: (0, 0)),
        ],
        out_specs=[
            pl.BlockSpec((blkj, Cout), lambda i: (i, 0)),
            pl.BlockSpec((2, Cout), lambda i: (0, 0)),
        ],
        out_shape=[
            jax.ShapeDtypeStruct((J, Cout), jnp.float32),
            jax.ShapeDtypeStruct((2, Cout), jnp.float32),
        ],
    )(y2_3d, a2, c2, w3t, b3)


def _affine_pass(z, a3, c3, blk):
    S, C = z.shape
    grid = (S // blk,)

    def body(z_ref, a_ref, c_ref, o_ref):
        o_ref[...] = jnp.maximum(z_ref[...] * a_ref[...] + c_ref[...], 0.0)

    return pl.pallas_call(
        body,
        grid=grid,
        in_specs=[
            pl.BlockSpec((blk, C), lambda i: (i, 0)),
            pl.BlockSpec((1, C), lambda i: (0, 0)),
            pl.BlockSpec((1, C), lambda i: (0, 0)),
        ],
        out_specs=pl.BlockSpec((blk, C), lambda i: (i, 0)),
        out_shape=jax.ShapeDtypeStruct((S, C), jnp.float32),
    )(z, a3, c3)


def _bn_affine(st, S, g, be):
    mean = st[0] / S
    var = st[1] / S - mean * mean
    a = g / jnp.sqrt(var + EPS)
    c = be - a * mean
    return a[None, :], c[None, :]


def kernel(q_points, s_points, s_feats, neighbor_indices,
           W1, b1, g1, be1, W2, b2, g2, be2, W3, b3, g3, be3):
    B, _, M = q_points.shape
    _, Ci, N = s_feats.shape
    K = neighbor_indices.shape[-1]
    C = Ci + 3
    J = B * M
    S = J * K

    table = jnp.concatenate(
        [s_feats.transpose(0, 2, 1), s_points.transpose(0, 2, 1)],
        axis=-1).reshape(B * N, C)
    idx = neighbor_indices.astype(jnp.int32) + \
        (jnp.arange(B, dtype=jnp.int32) * N)[:, None, None]
    idx = idx.transpose(2, 0, 1).reshape(-1)
    qf = q_points.transpose(0, 2, 1).reshape(J, 3)
    qpad = jnp.zeros((J, C), jnp.float32).at[:, Ci:].set(qf)

    x = _sc_gather(table, idx)

    y1, st1 = _stage1_pass(x, qpad, W1.T, b1[None, :], 2048)
    a1, c1 = _bn_affine(st1, S, g1, be1)

    y2, st2 = _stage2_pass(y1, a1, c1, W2.T, b2[None, :], 2048)
    a2, c2 = _bn_affine(st2, S, g2, be2)

    z, st3 = _final_pass(y2.reshape(K, J, C), a2, c2, W3.T, b3[None, :], 512)
    a3, c3 = _bn_affine(st3, S, g3, be3)

    out = _affine_pass(z, a3, c3, 2048)
    return out.reshape(B, M, -1).transpose(0, 2, 1)


# speedup vs baseline: 512.9210x; 1.0704x over previous
import functools

import jax
import jax.numpy as jnp
from jax import lax
from jax.experimental import pallas as pl
from jax.experimental.pallas import tpu as pltpu
from jax.experimental.pallas import tpu_sc as plsc

EPS = 1e-5
NW = 32
GCH = 1024
GSUB = 128


def _sc_gather(table, idx):
    R, C = table.shape
    S = idx.shape[0]
    per_w = S // NW
    ngroups = per_w // GCH
    nsub = GCH // GSUB

    mesh = plsc.VectorSubcoreMesh(core_axis_name="c", subcore_axis_name="s")

    @functools.partial(
        pl.kernel,
        out_type=jax.ShapeDtypeStruct((S, C), jnp.float32),
        mesh=mesh,
        compiler_params=pltpu.CompilerParams(use_tc_tiling_on_sc=False),
        scratch_types=[
            pltpu.VMEM((per_w,), jnp.int32),
            pltpu.VMEM((GCH, C), jnp.float32),
            pltpu.SemaphoreType.DMA,
        ],
    )
    def gk(table_hbm, idx_hbm, out_hbm, idx_v, rows_v, sem):
        wid = lax.axis_index("s") * 2 + lax.axis_index("c")
        base = wid * per_w
        pltpu.sync_copy(idx_hbm.at[pl.ds(base, per_w)], idx_v)

        def body(c, carry):
            cb = c * GCH
            handles = []
            for j in range(nsub):
                handles.append(pltpu.async_copy(
                    table_hbm.at[idx_v.at[pl.ds(cb + j * GSUB, GSUB)]],
                    rows_v.at[pl.ds(j * GSUB, GSUB)],
                    sem))
            for h in handles:
                h.wait()
            pltpu.sync_copy(rows_v, out_hbm.at[pl.ds(base + cb, GCH)])
            return carry

        lax.fori_loop(0, ngroups, body, 0)

    return gk(table, idx)


def _stats_outputs(Cout):
    return (
        pl.BlockSpec((2, Cout), lambda i: (0, 0)),
        jax.ShapeDtypeStruct((2, Cout), jnp.float32),
    )


def _accum_stats(st_ref, y):
    @pl.when(pl.program_id(0) == 0)
    def _():
        st_ref[...] = jnp.zeros_like(st_ref)

    st_ref[...] += jnp.concatenate(
        [jnp.sum(y, axis=0, keepdims=True),
         jnp.sum(y * y, axis=0, keepdims=True)], axis=0)


def _stage1_pass(x, qpad, w1t, b1, blk, out_dtype):
    S, Cin = x.shape
    Cout = w1t.shape[1]
    grid = (S // blk,)
    nq = qpad.shape[0] // blk

    def body(x_ref, p_ref, w_ref, b_ref, y_ref, st_ref):
        h = x_ref[...].astype(jnp.float32) - p_ref[...]
        y = jnp.dot(h, w_ref[...], preferred_element_type=jnp.float32) + b_ref[...]
        y_ref[...] = y.astype(out_dtype)
        _accum_stats(st_ref, y)

    st_spec, st_shape = _stats_outputs(Cout)
    return pl.pallas_call(
        body,
        grid=grid,
        in_specs=[
            pl.BlockSpec((blk, Cin), lambda i: (i, 0)),
            pl.BlockSpec((blk, Cin), lambda i: (i % nq, 0)),
            pl.BlockSpec((Cin, Cout), lambda i: (0, 0)),
            pl.BlockSpec((1, Cout), lambda i: (0, 0)),
        ],
        out_specs=[pl.BlockSpec((blk, Cout), lambda i: (i, 0)), st_spec],
        out_shape=[jax.ShapeDtypeStruct((S, Cout), out_dtype), st_shape],
    )(x, qpad, w1t, b1)


def _stage2_pass(y1, a1, c1, w2t, b2, blk, out_dtype):
    S, Cin = y1.shape
    Cout = w2t.shape[1]
    grid = (S // blk,)

    def body(x_ref, a_ref, c_ref, w_ref, b_ref, y_ref, st_ref):
        h = jnp.maximum(x_ref[...].astype(jnp.float32) * a_ref[...] + c_ref[...], 0.0)
        y = jnp.dot(h, w_ref[...], preferred_element_type=jnp.float32) + b_ref[...]
        y_ref[...] = y.astype(out_dtype)
        _accum_stats(st_ref, y)

    st_spec, st_shape = _stats_outputs(Cout)
    return pl.pallas_call(
        body,
        grid=grid,
        in_specs=[
            pl.BlockSpec((blk, Cin), lambda i: (i, 0)),
            pl.BlockSpec((1, Cin), lambda i: (0, 0)),
            pl.BlockSpec((1, Cin), lambda i: (0, 0)),
            pl.BlockSpec((Cin, Cout), lambda i: (0, 0)),
            pl.BlockSpec((1, Cout), lambda i: (0, 0)),
        ],
        out_specs=[pl.BlockSpec((blk, Cout), lambda i: (i, 0)), st_spec],
        out_shape=[jax.ShapeDtypeStruct((S, Cout), out_dtype), st_shape],
    )(y1, a1, c1, w2t, b2)


def _final_pass(y2_3d, a2, c2, w3t, b3, blkj):
    K, J, Cin = y2_3d.shape
    Cout = w3t.shape[1]
    grid = (J // blkj,)

    def body(y_ref, a_ref, c_ref, w_ref, b_ref, z_ref, st_ref):
        h = jnp.maximum(y_ref[...].astype(jnp.float32) * a_ref[...] + c_ref[...], 0.0)
        y3 = jnp.dot(h.reshape(K * blkj, Cin), w_ref[...],
                     preferred_element_type=jnp.float32) + b_ref[...]
        z_ref[...] = jnp.max(y3.reshape(K, blkj, Cout), axis=0)
        _accum_stats(st_ref, y3)

    return pl.pallas_call(
        body,
        grid=grid,
        in_specs=[
            pl.BlockSpec((K, blkj, Cin), lambda i: (0, i, 0)),
            pl.BlockSpec((1, Cin), lambda i: (0, 0)),
            pl.BlockSpec((1, Cin), lambda i: (0, 0)),
            pl.BlockSpec((Cin, Cout), lambda i: (0, 0)),
            pl.BlockSpec((1, Cout), lambda i: (0, 0)),
        ],
        out_specs=[
            pl.BlockSpec((blkj, Cout), lambda i: (i, 0)),
            pl.BlockSpec((2, Cout), lambda i: (0, 0)),
        ],
        out_shape=[
            jax.ShapeDtypeStruct((J, Cout), jnp.float32),
            jax.ShapeDtypeStruct((2, Cout), jnp.float32),
        ],
    )(y2_3d, a2, c2, w3t, b3)


def _affine_pass(z, a3, c3, blk):
    S, C = z.shape
    grid = (S // blk,)

    def body(z_ref, a_ref, c_ref, o_ref):
        o_ref[...] = jnp.maximum(z_ref[...] * a_ref[...] + c_ref[...], 0.0)

    return pl.pallas_call(
        body,
        grid=grid,
        in_specs=[
            pl.BlockSpec((blk, C), lambda i: (i, 0)),
            pl.BlockSpec((1, C), lambda i: (0, 0)),
            pl.BlockSpec((1, C), lambda i: (0, 0)),
        ],
        out_specs=pl.BlockSpec((blk, C), lambda i: (i, 0)),
        out_shape=jax.ShapeDtypeStruct((S, C), jnp.float32),
    )(z, a3, c3)


def _bn_affine(st, S, g, be):
    mean = st[0] / S
    var = st[1] / S - mean * mean
    a = g / jnp.sqrt(var + EPS)
    c = be - a * mean
    return a[None, :], c[None, :]


def kernel(q_points, s_points, s_feats, neighbor_indices,
           W1, b1, g1, be1, W2, b2, g2, be2, W3, b3, g3, be3):
    B, _, M = q_points.shape
    _, Ci, N = s_feats.shape
    K = neighbor_indices.shape[-1]
    C = Ci + 3
    J = B * M
    S = J * K

    table = jnp.concatenate(
        [s_feats.transpose(0, 2, 1), s_points.transpose(0, 2, 1)],
        axis=-1).reshape(B * N, C)
    idx = neighbor_indices.astype(jnp.int32) + \
        (jnp.arange(B, dtype=jnp.int32) * N)[:, None, None]
    idx = idx.transpose(2, 0, 1).reshape(-1)
    qf = q_points.transpose(0, 2, 1).reshape(J, 3)
    qpad = jnp.zeros((J, C), jnp.float32).at[:, Ci:].set(qf)

    x = _sc_gather(table, idx)

    y1, st1 = _stage1_pass(x, qpad, W1.T, b1[None, :], 2048, jnp.bfloat16)
    a1, c1 = _bn_affine(st1, S, g1, be1)

    y2, st2 = _stage2_pass(y1, a1, c1, W2.T, b2[None, :], 2048, jnp.bfloat16)
    a2, c2 = _bn_affine(st2, S, g2, be2)

    z, st3 = _final_pass(y2.reshape(K, J, C), a2, c2, W3.T, b3[None, :], 512)
    a3, c3 = _bn_affine(st3, S, g3, be3)

    out = _affine_pass(z, a3, c3, 2048)
    return out.reshape(B, M, -1).transpose(0, 2, 1)


# speedup vs baseline: 664.5515x; 1.2956x over previous
import functools

import jax
import jax.numpy as jnp
from jax import lax
from jax.experimental import pallas as pl
from jax.experimental.pallas import tpu as pltpu
from jax.experimental.pallas import tpu_sc as plsc

EPS = 1e-5
NW = 32
GCH = 1024
GSUB = 128


def _sc_gather(table, idx):
    R, C = table.shape
    S = idx.shape[0]
    per_w = S // NW
    ngroups = per_w // GCH
    nsub = GCH // GSUB

    mesh = plsc.VectorSubcoreMesh(core_axis_name="c", subcore_axis_name="s")

    @functools.partial(
        pl.kernel,
        out_type=jax.ShapeDtypeStruct((S, C), jnp.float32),
        mesh=mesh,
        compiler_params=pltpu.CompilerParams(use_tc_tiling_on_sc=False),
        scratch_types=[
            pltpu.VMEM((per_w,), jnp.int32),
            pltpu.VMEM((GCH, C), jnp.float32),
            pltpu.SemaphoreType.DMA,
        ],
    )
    def gk(table_hbm, idx_hbm, out_hbm, idx_v, rows_v, sem):
        wid = lax.axis_index("s") * 2 + lax.axis_index("c")
        base = wid * per_w
        pltpu.sync_copy(idx_hbm.at[pl.ds(base, per_w)], idx_v)

        def body(c, carry):
            cb = c * GCH
            handles = []
            for j in range(nsub):
                handles.append(pltpu.async_copy(
                    table_hbm.at[idx_v.at[pl.ds(cb + j * GSUB, GSUB)]],
                    rows_v.at[pl.ds(j * GSUB, GSUB)],
                    sem))
            for h in handles:
                h.wait()
            pltpu.sync_copy(rows_v, out_hbm.at[pl.ds(base + cb, GCH)])
            return carry

        lax.fori_loop(0, ngroups, body, 0)

    return gk(table, idx)


def _stats_outputs(Cout):
    return (
        pl.BlockSpec((2, Cout), lambda i: (0, 0)),
        jax.ShapeDtypeStruct((2, Cout), jnp.float32),
    )


def _accum_stats(st_ref, y):
    @pl.when(pl.program_id(0) == 0)
    def _():
        st_ref[...] = jnp.zeros_like(st_ref)

    st_ref[...] += jnp.concatenate(
        [jnp.sum(y, axis=0, keepdims=True),
         jnp.sum(y * y, axis=0, keepdims=True)], axis=0)


def _stage1_pass(x, qpad, w1t, b1, blk, out_dtype):
    S, Cin = x.shape
    Cout = w1t.shape[1]
    grid = (S // blk,)
    nq = qpad.shape[0] // blk

    def body(x_ref, p_ref, w_ref, b_ref, y_ref, st_ref):
        h = x_ref[...].astype(jnp.float32) - p_ref[...]
        y = jnp.dot(h, w_ref[...], preferred_element_type=jnp.float32) + b_ref[...]
        y_ref[...] = y.astype(out_dtype)
        _accum_stats(st_ref, y)

    st_spec, st_shape = _stats_outputs(Cout)
    return pl.pallas_call(
        body,
        grid=grid,
        in_specs=[
            pl.BlockSpec((blk, Cin), lambda i: (i, 0)),
            pl.BlockSpec((blk, Cin), lambda i: (i % nq, 0)),
            pl.BlockSpec((Cin, Cout), lambda i: (0, 0)),
            pl.BlockSpec((1, Cout), lambda i: (0, 0)),
        ],
        out_specs=[pl.BlockSpec((blk, Cout), lambda i: (i, 0)), st_spec],
        out_shape=[jax.ShapeDtypeStruct((S, Cout), out_dtype), st_shape],
    )(x, qpad, w1t, b1)


def _stage2_pass(y1, a1, c1, w2t, b2, blk, out_dtype):
    S, Cin = y1.shape
    Cout = w2t.shape[1]
    grid = (S // blk,)

    def body(x_ref, a_ref, c_ref, w_ref, b_ref, y_ref, st_ref):
        h = jnp.maximum(x_ref[...].astype(jnp.float32) * a_ref[...] + c_ref[...], 0.0)
        y = jnp.dot(h, w_ref[...], preferred_element_type=jnp.float32) + b_ref[...]
        y_ref[...] = y.astype(out_dtype)
        _accum_stats(st_ref, y)

    st_spec, st_shape = _stats_outputs(Cout)
    return pl.pallas_call(
        body,
        grid=grid,
        in_specs=[
            pl.BlockSpec((blk, Cin), lambda i: (i, 0)),
            pl.BlockSpec((1, Cin), lambda i: (0, 0)),
            pl.BlockSpec((1, Cin), lambda i: (0, 0)),
            pl.BlockSpec((Cin, Cout), lambda i: (0, 0)),
            pl.BlockSpec((1, Cout), lambda i: (0, 0)),
        ],
        out_specs=[pl.BlockSpec((blk, Cout), lambda i: (i, 0)), st_spec],
        out_shape=[jax.ShapeDtypeStruct((S, Cout), out_dtype), st_shape],
    )(y1, a1, c1, w2t, b2)


def _final_pass(y2_3d, a2, c2, w3t, b3, blkj):
    K, J, Cin = y2_3d.shape
    Cout = w3t.shape[1]
    grid = (J // blkj,)

    def body(y_ref, a_ref, c_ref, w_ref, b_ref, z_ref, st_ref):
        h = jnp.maximum(y_ref[...].astype(jnp.float32) * a_ref[...] + c_ref[...], 0.0)
        y3 = jnp.dot(h.reshape(K * blkj, Cin), w_ref[...],
                     preferred_element_type=jnp.float32) + b_ref[...]
        z_ref[...] = jnp.max(y3.reshape(K, blkj, Cout), axis=0)
        _accum_stats(st_ref, y3)

    return pl.pallas_call(
        body,
        grid=grid,
        in_specs=[
            pl.BlockSpec((K, blkj, Cin), lambda i: (0, i, 0)),
            pl.BlockSpec((1, Cin), lambda i: (0, 0)),
            pl.BlockSpec((1, Cin), lambda i: (0, 0)),
            pl.BlockSpec((Cin, Cout), lambda i: (0, 0)),
            pl.BlockSpec((1, Cout), lambda i: (0, 0)),
        ],
        out_specs=[
            pl.BlockSpec((blkj, Cout), lambda i: (i, 0)),
            pl.BlockSpec((2, Cout), lambda i: (0, 0)),
        ],
        out_shape=[
            jax.ShapeDtypeStruct((J, Cout), jnp.float32),
            jax.ShapeDtypeStruct((2, Cout), jnp.float32),
        ],
    )(y2_3d, a2, c2, w3t, b3)


def _affine_pass(z, a3, c3, blk):
    S, C = z.shape
    grid = (S // blk,)

    def body(z_ref, a_ref, c_ref, o_ref):
        o_ref[...] = jnp.maximum(z_ref[...] * a_ref[...] + c_ref[...], 0.0)

    return pl.pallas_call(
        body,
        grid=grid,
        in_specs=[
            pl.BlockSpec((blk, C), lambda i: (i, 0)),
            pl.BlockSpec((1, C), lambda i: (0, 0)),
            pl.BlockSpec((1, C), lambda i: (0, 0)),
        ],
        out_specs=pl.BlockSpec((blk, C), lambda i: (i, 0)),
        out_shape=jax.ShapeDtypeStruct((S, C), jnp.float32),
    )(z, a3, c3)


def _bn_affine(st, S, g, be):
    mean = st[0] / S
    var = st[1] / S - mean * mean
    a = g / jnp.sqrt(var + EPS)
    c = be - a * mean
    return a[None, :], c[None, :]


def kernel(q_points, s_points, s_feats, neighbor_indices,
           W1, b1, g1, be1, W2, b2, g2, be2, W3, b3, g3, be3):
    B, _, M = q_points.shape
    _, Ci, N = s_feats.shape
    K = neighbor_indices.shape[-1]
    C = Ci + 3
    J = B * M
    S = J * K

    table = jnp.concatenate(
        [s_feats.transpose(0, 2, 1), s_points.transpose(0, 2, 1)],
        axis=-1).reshape(B * N, C)
    idx = neighbor_indices.astype(jnp.int32) + \
        (jnp.arange(B, dtype=jnp.int32) * N)[:, None, None]
    idx = idx.transpose(2, 0, 1).reshape(-1)
    qf = q_points.transpose(0, 2, 1).reshape(J, 3)
    qpad = jnp.zeros((J, C), jnp.float32).at[:, Ci:].set(qf)

    x = _sc_gather(table, idx)

    y1, st1 = _stage1_pass(x, qpad, W1.T, b1[None, :], 8192, jnp.bfloat16)
    a1, c1 = _bn_affine(st1, S, g1, be1)

    y2, st2 = _stage2_pass(y1, a1, c1, W2.T, b2[None, :], 8192, jnp.bfloat16)
    a2, c2 = _bn_affine(st2, S, g2, be2)

    z, st3 = _final_pass(y2.reshape(K, J, C), a2, c2, W3.T, b3[None, :], 512)
    a3, c3 = _bn_affine(st3, S, g3, be3)

    out = _affine_pass(z, a3, c3, 2048)
    return out.reshape(B, M, -1).transpose(0, 2, 1)
